# baseline (speedup 1.0000x reference)
import functools

import jax
import jax.numpy as jnp
from jax import lax
from jax.experimental import pallas as pl
from jax.experimental.pallas import tpu as pltpu
from jax.experimental.pallas import tpu_sc as plsc

N_NODES = 10000
N_EDGES = 320000
D = 128
HID = 256

NC = 2
NS = 16
NW = NC * NS
EPW = N_EDGES // NW
CH = 80
NCH = EPW // CH
RB = 624
TAIL = N_NODES - NS * RB

_mesh = plsc.VectorSubcoreMesh(
    core_axis_name="c", subcore_axis_name="s", num_cores=NC, num_subcores=NS)


@functools.partial(
    pl.kernel,
    out_type=(jax.ShapeDtypeStruct((N_EDGES, D), jnp.float32),
              jax.ShapeDtypeStruct((N_EDGES, D), jnp.float32)),
    mesh=_mesh,
    scratch_types=[
        pltpu.VMEM((CH,), jnp.int32),
        pltpu.VMEM((CH,), jnp.int32),
        pltpu.VMEM((CH, D), jnp.float32),
        pltpu.VMEM((CH, D), jnp.float32),
        pltpu.SemaphoreType.DMA,
        pltpu.SemaphoreType.DMA,
    ],
)
def _sc_gather(v_hbm, s_hbm, r_hbm, outs, outr,
               si_v, ri_v, srow_v, rrow_v, sem1, sem2):
    wid = lax.axis_index("s") * NC + lax.axis_index("c")
    base = wid * EPW

    def body(c, carry):
        off = base + c * CH
        pltpu.sync_copy(s_hbm.at[pl.ds(off, CH)], si_v)
        pltpu.sync_copy(r_hbm.at[pl.ds(off, CH)], ri_v)
        cp1 = pltpu.async_copy(v_hbm.at[si_v], srow_v, sem1)
        cp2 = pltpu.async_copy(v_hbm.at[ri_v], rrow_v, sem2)
        cp1.wait()
        pltpu.sync_copy(srow_v, outs.at[pl.ds(off, CH)])
        cp2.wait()
        pltpu.sync_copy(rrow_v, outr.at[pl.ds(off, CH)])
        return carry

    lax.fori_loop(0, NCH, body, 0)


@functools.partial(
    pl.kernel,
    out_type=(jax.ShapeDtypeStruct((NC * N_NODES, D), jnp.float32),
              jax.ShapeDtypeStruct((NC * N_NODES, D), jnp.float32)),
    mesh=_mesh,
    scratch_types=[
        pltpu.VMEM((CH,), jnp.int32),
        pltpu.VMEM((CH, D), jnp.float32),
        pltpu.VMEM((CH, D), jnp.float32),
        pltpu.VMEM_SHARED((N_NODES, D), jnp.float32),
    ],
)
def _sc_scatter(emb_hbm, ridx_hbm, zrow_hbm, ones_hbm,
                sums_hbm, cnts_hbm,
                idx_v, row_v, ones_v, acc):
    cid = lax.axis_index("c")
    sid = lax.axis_index("s")
    wid = sid * NC + cid
    obase = cid * N_NODES

    def zero_acc():
        pltpu.sync_copy(zrow_hbm.at[pl.ds(0, RB)], acc.at[pl.ds(sid * RB, RB)])

        @pl.when(sid == NS - 1)
        def _zero_tail():
            pltpu.sync_copy(zrow_hbm.at[pl.ds(RB, TAIL)],
                            acc.at[pl.ds(NS * RB, TAIL)])

    def dump(out_hbm):
        pltpu.sync_copy(acc.at[pl.ds(sid * RB, RB)],
                        out_hbm.at[pl.ds(obase + sid * RB, RB)])

        @pl.when(sid == NS - 1)
        def _dump_tail():
            pltpu.sync_copy(acc.at[pl.ds(NS * RB, TAIL)],
                            out_hbm.at[pl.ds(obase + NS * RB, TAIL)])

    zero_acc()
    plsc.subcore_barrier()

    def body(c, carry):
        off = wid * EPW + c * CH
        pltpu.sync_copy(ridx_hbm.at[pl.ds(off, CH)], idx_v)
        pltpu.sync_copy(emb_hbm.at[pl.ds(off, CH)], row_v)
        pltpu.sync_copy(row_v, acc.at[idx_v], add=True)
        return carry

    lax.fori_loop(0, NCH, body, 0)
    plsc.subcore_barrier()
    dump(sums_hbm)

    zero_acc()
    pltpu.sync_copy(ones_hbm, ones_v)
    plsc.subcore_barrier()

    def body2(c, carry):
        off = wid * EPW + c * CH
        pltpu.sync_copy(ridx_hbm.at[pl.ds(off, CH)], idx_v)
        pltpu.sync_copy(ones_v, acc.at[idx_v], add=True)
        return carry

    lax.fori_loop(0, NCH, body2, 0)
    plsc.subcore_barrier()
    dump(cnts_hbm)


def _gelu(x):
    return x * 0.5 * (1.0 + lax.erf(x * 0.7071067811865476))


BE = 1280


def _edge_mlp_body(s_ref, r_ref, e_ref, w1s, w1r, w1e, b1, w2, b2, o_ref):
    bf = jnp.bfloat16
    x = (jnp.dot(s_ref[...].astype(bf), w1s[...].astype(bf),
                 preferred_element_type=jnp.float32)
         + jnp.dot(r_ref[...].astype(bf), w1r[...].astype(bf),
                   preferred_element_type=jnp.float32)
         + jnp.dot(e_ref[...].astype(bf), w1e[...].astype(bf),
                   preferred_element_type=jnp.float32)
         + b1[...])
    h = _gelu(x)
    o_ref[...] = jnp.dot(h.astype(bf), w2[...].astype(bf),
                         preferred_element_type=jnp.float32) + b2[...]


def _tc_edge_mlp(S, R, E2, w1s, w1r, w1e, b1, w2, b2):
    full = lambda shape: pl.BlockSpec(shape, lambda i: (0,) * len(shape))
    return pl.pallas_call(
        _edge_mlp_body,
        grid=(N_EDGES // BE,),
        in_specs=[
            pl.BlockSpec((BE, D), lambda i: (i, 0)),
            pl.BlockSpec((BE, D), lambda i: (i, 0)),
            pl.BlockSpec((BE, D), lambda i: (i, 0)),
            full((D, HID)), full((D, HID)), full((D, HID)),
            full((1, HID)), full((HID, D)), full((1, D)),
        ],
        out_specs=pl.BlockSpec((BE, D), lambda i: (i, 0)),
        out_shape=jax.ShapeDtypeStruct((N_EDGES, D), jnp.float32),
    )(S, R, E2, w1s, w1r, w1e, b1, w2, b2)


BN = 400


def _node_mlp_body(v_ref, s0, s1, c0, c1, w1v, w1e, b1, w2, b2, o_ref):
    cnt = c0[...][:, :1] + c1[...][:, :1]
    es = (s0[...] + s1[...]) / jnp.maximum(cnt, 1.0)
    x = (jnp.dot(v_ref[...], w1v[...], preferred_element_type=jnp.float32)
         + jnp.dot(es, w1e[...], preferred_element_type=jnp.float32)
         + b1[...])
    h = _gelu(x)
    o_ref[...] = jnp.dot(h, w2[...], preferred_element_type=jnp.float32) + b2[...]


def _tc_node_mlp(V2, s0, s1, c0, c1, w1v, w1e, b1, w2, b2):
    full = lambda shape: pl.BlockSpec(shape, lambda i: (0,) * len(shape))
    return pl.pallas_call(
        _node_mlp_body,
        grid=(N_NODES // BN,),
        in_specs=[
            pl.BlockSpec((BN, D), lambda i: (i, 0)),
            pl.BlockSpec((BN, D), lambda i: (i, 0)),
            pl.BlockSpec((BN, D), lambda i: (i, 0)),
            pl.BlockSpec((BN, D), lambda i: (i, 0)),
            pl.BlockSpec((BN, D), lambda i: (i, 0)),
            full((D, HID)), full((D, HID)),
            full((1, HID)), full((HID, D)), full((1, D)),
        ],
        out_specs=pl.BlockSpec((BN, D), lambda i: (i, 0)),
        out_shape=jax.ShapeDtypeStruct((N_NODES, D), jnp.float32),
    )(V2, s0, s1, c0, c1, w1v, w1e, b1, w2, b2)


def kernel(V, E, edges, fe_W1, fe_b1, fe_W2, fe_b2, fn_W1, fn_b1, fn_W2, fn_b2):
    V2 = V[0]
    E2 = E[0]
    eidx = edges[0].astype(jnp.int32)
    sidx = eidx[:, 0]
    ridx = eidx[:, 1]

    S, R = _sc_gather(V2, sidx, ridx)

    emb = _tc_edge_mlp(
        S, R, E2,
        fe_W1[:D], fe_W1[D:2 * D], fe_W1[2 * D:],
        fe_b1.reshape(1, HID), fe_W2, fe_b2.reshape(1, D))

    zrow = jnp.zeros((RB + TAIL, D), jnp.float32)
    ones = jnp.ones((CH, D), jnp.float32)
    sums_f, cnts_f = _sc_scatter(emb, ridx, zrow, ones)
    sums = sums_f.reshape(NC, N_NODES, D)
    cnts = cnts_f.reshape(NC, N_NODES, D)

    nodes = _tc_node_mlp(
        V2, sums[0], sums[1], cnts[0], cnts[1],
        fn_W1[:D], fn_W1[D:],
        fn_b1.reshape(1, HID), fn_W2, fn_b2.reshape(1, D))

    return (nodes[None], emb[None])


# speedup vs baseline: 1.4130x; 1.4130x over previous
import functools

import jax
import jax.numpy as jnp
from jax import lax
from jax.experimental import pallas as pl
from jax.experimental.pallas import tpu as pltpu
from jax.experimental.pallas import tpu_sc as plsc

N_NODES = 10000
N_EDGES = 320000
D = 128
HID = 256

NC = 2
NS = 16
NW = NC * NS
EPW = N_EDGES // NW
CH = 80
NCH = EPW // CH
NBUF = 4
RB = 624
TAIL = N_NODES - NS * RB

_mesh = plsc.VectorSubcoreMesh(
    core_axis_name="c", subcore_axis_name="s", num_cores=NC, num_subcores=NS)


@functools.partial(
    pl.kernel,
    out_type=(jax.ShapeDtypeStruct((N_EDGES, D), jnp.float32),
              jax.ShapeDtypeStruct((N_EDGES, D), jnp.float32)),
    mesh=_mesh,
    scratch_types=[
        pltpu.VMEM((NCH, CH), jnp.int32),
        pltpu.VMEM((NCH, CH), jnp.int32),
        pltpu.VMEM((NBUF, CH, D), jnp.float32),
        pltpu.VMEM((NBUF, CH, D), jnp.float32),
        [pltpu.SemaphoreType.DMA] * NBUF,
        [pltpu.SemaphoreType.DMA] * NBUF,
        [pltpu.SemaphoreType.DMA] * NBUF,
        [pltpu.SemaphoreType.DMA] * NBUF,
    ],
)
def _sc_gather(v_hbm, sidx_hbm, ridx_hbm,
               outs, outr,
               si_all, ri_all, bufs, bufr,
               sg, rg, sw, rw):
    cid = lax.axis_index("c")
    sid = lax.axis_index("s")
    wid = sid * NC + cid
    base = wid * EPW

    pltpu.sync_copy(sidx_hbm.at[wid], si_all)
    pltpu.sync_copy(ridx_hbm.at[wid], ri_all)

    def g_descs(c, b):
        return (pltpu.make_async_copy(v_hbm.at[si_all.at[c]], bufs.at[b], sg[b]),
                pltpu.make_async_copy(v_hbm.at[ri_all.at[c]], bufr.at[b], rg[b]))

    def w_descs(c, b):
        dst = pl.ds(base + c * CH, CH)
        return (pltpu.make_async_copy(bufs.at[b], outs.at[dst], sw[b]),
                pltpu.make_async_copy(bufr.at[b], outr.at[dst], rw[b]))

    def start_gathers(c, b):
        d1, d2 = g_descs(c, b)
        d1.start()
        d2.start()

    def visit(c, b):
        b2 = (b + 2) % NBUF

        @pl.when(c >= 2)
        def _retire():
            d1, d2 = w_descs(c - 2, b2)
            d1.wait()
            d2.wait()

        @pl.when(c + 2 < NCH)
        def _prefetch():
            start_gathers(c + 2, b2)

        d1, d2 = g_descs(c, b)
        d1.wait()
        d2.wait()
        d1, d2 = w_descs(c, b)
        d1.start()
        d2.start()

    start_gathers(0, 0)
    start_gathers(1, 1)

    def body(g, carry):
        c0 = g * NBUF
        for b in range(NBUF):
            visit(c0 + b, b)
        return carry

    lax.fori_loop(0, (NCH - 1) // NBUF, body, 0)
    visit(NCH - 1, (NCH - 1) % NBUF)

    for c in (NCH - 2, NCH - 1):
        b = c % NBUF
        d1, d2 = w_descs(c, b)
        d1.wait()
        d2.wait()


CHS = 40
NH = 2
NCHH = EPW // CHS // NH
NBS = 3


@functools.partial(
    pl.kernel,
    out_type=(jax.ShapeDtypeStruct((NC * N_NODES, D), jnp.float32),
              jax.ShapeDtypeStruct((NC * N_NODES, D), jnp.float32)),
    mesh=_mesh,
    scratch_types=[
        pltpu.VMEM((NCHH, CHS), jnp.int32),
        pltpu.VMEM((NBS, CHS, D), jnp.float32),
        pltpu.VMEM((CHS, D), jnp.float32),
        pltpu.VMEM_SHARED((N_NODES, D), jnp.float32),
        [pltpu.SemaphoreType.DMA] * NBS,
        [pltpu.SemaphoreType.DMA] * NBS,
        [pltpu.SemaphoreType.DMA] * 2,
    ],
)
def _sc_scatter(emb_hbm, ridx_hbm, zrow_hbm, ones_hbm,
                sums_hbm, cnts_hbm,
                ri_half, bufe, ones_v, acc, le, sa, ca):
    cid = lax.axis_index("c")
    sid = lax.axis_index("s")
    wid = sid * NC + cid
    base = wid * EPW
    obase = cid * N_NODES

    pltpu.sync_copy(ones_hbm, ones_v)

    def zero_acc():
        pltpu.sync_copy(zrow_hbm.at[pl.ds(0, RB)], acc.at[pl.ds(sid * RB, RB)])

        @pl.when(sid == NS - 1)
        def _zero_tail():
            pltpu.sync_copy(zrow_hbm.at[pl.ds(RB, TAIL)],
                            acc.at[pl.ds(NS * RB, TAIL)])

    def dump(out_hbm):
        pltpu.sync_copy(acc.at[pl.ds(sid * RB, RB)],
                        out_hbm.at[pl.ds(obase + sid * RB, RB)])

        @pl.when(sid == NS - 1)
        def _dump_tail():
            pltpu.sync_copy(acc.at[pl.ds(NS * RB, TAIL)],
                            out_hbm.at[pl.ds(obase + NS * RB, TAIL)])

    def l_desc(h, c, b):
        src = pl.ds(base + (h * NCHH + c) * CHS, CHS)
        return pltpu.make_async_copy(emb_hbm.at[src], bufe.at[b], le[b])

    def a_desc(c, b):
        return pltpu.make_async_copy(bufe.at[b], acc.at[ri_half.at[c]], sa[b])

    zero_acc()
    plsc.subcore_barrier()

    for h in range(NH):
        pltpu.sync_copy(ridx_hbm.at[wid, h], ri_half)

        def visit(c, b, h=h):
            bp = (b + 2) % NBS

            @pl.when(c >= 1)
            def _retire():
                a_desc(c - 1, bp).wait()

            @pl.when(c + 2 < NCHH)
            def _prefetch():
                l_desc(h, c + 2, bp).start()

            l_desc(h, c, b).wait()
            a_desc(c, b).start(add=True)

        l_desc(h, 0, 0).start()
        l_desc(h, 1, 1).start()

        def body(g, carry, visit=visit):
            c0 = g * NBS
            for b in range(NBS):
                visit(c0 + b, b)
            return carry

        nfull = (NCHH - 2) // NBS
        lax.fori_loop(0, nfull, body, 0)
        for c in range(nfull * NBS, NCHH):
            visit(c, c % NBS)
        a_desc(NCHH - 1, (NCHH - 1) % NBS).wait()

    plsc.subcore_barrier()
    dump(sums_hbm)

    zero_acc()
    plsc.subcore_barrier()

    def c_desc(c, k):
        return pltpu.make_async_copy(ones_v, acc.at[ri_half.at[c]], ca[k])

    for h in range(NH):
        pltpu.sync_copy(ridx_hbm.at[wid, h], ri_half)

        def cvisit(c, k):
            @pl.when(c >= 2)
            def _retire():
                c_desc(c - 2, k).wait()

            c_desc(c, k).start(add=True)

        def body2(g, carry, cvisit=cvisit):
            c0 = g * 2
            cvisit(c0, 0)
            cvisit(c0 + 1, 1)
            return carry

        lax.fori_loop(0, NCHH // 2, body2, 0)
        for c in range(2 * (NCHH // 2), NCHH):
            cvisit(c, c % 2)
        for c in (NCHH - 2, NCHH - 1):
            c_desc(c, c % 2).wait()

    plsc.subcore_barrier()
    dump(cnts_hbm)


def _gelu(x):
    return x * 0.5 * (1.0 + lax.erf(x * 0.7071067811865476))


BE = 1280


def _edge_mlp_body(s_ref, r_ref, e_ref, w1s, w1r, w1e, b1, w2, b2, o_ref):
    x = (jnp.dot(s_ref[...], w1s[...], preferred_element_type=jnp.float32)
         + jnp.dot(r_ref[...], w1r[...], preferred_element_type=jnp.float32)
         + jnp.dot(e_ref[...], w1e[...], preferred_element_type=jnp.float32)
         + b1[...])
    h = _gelu(x)
    o_ref[...] = jnp.dot(h, w2[...], preferred_element_type=jnp.float32) + b2[...]


def _tc_edge_mlp(S, R, E2, w1s, w1r, w1e, b1, w2, b2):
    full = lambda shape: pl.BlockSpec(shape, lambda i: (0,) * len(shape))
    return pl.pallas_call(
        _edge_mlp_body,
        grid=(N_EDGES // BE,),
        in_specs=[
            pl.BlockSpec((BE, D), lambda i: (i, 0)),
            pl.BlockSpec((BE, D), lambda i: (i, 0)),
            pl.BlockSpec((BE, D), lambda i: (i, 0)),
            full((D, HID)), full((D, HID)), full((D, HID)),
            full((1, HID)), full((HID, D)), full((1, D)),
        ],
        out_specs=pl.BlockSpec((BE, D), lambda i: (i, 0)),
        out_shape=jax.ShapeDtypeStruct((N_EDGES, D), jnp.float32),
    )(S, R, E2, w1s, w1r, w1e, b1, w2, b2)


BN = 400


def _node_mlp_body(v_ref, s0, s1, c0, c1, w1v, w1e, b1, w2, b2, o_ref):
    cnt = c0[...][:, :1] + c1[...][:, :1]
    es = (s0[...] + s1[...]) / jnp.maximum(cnt, 1.0)
    x = (jnp.dot(v_ref[...], w1v[...], preferred_element_type=jnp.float32)
         + jnp.dot(es, w1e[...], preferred_element_type=jnp.float32)
         + b1[...])
    h = _gelu(x)
    o_ref[...] = jnp.dot(h, w2[...], preferred_element_type=jnp.float32) + b2[...]


def _tc_node_mlp(V2, s0, s1, c0, c1, w1v, w1e, b1, w2, b2):
    full = lambda shape: pl.BlockSpec(shape, lambda i: (0,) * len(shape))
    return pl.pallas_call(
        _node_mlp_body,
        grid=(N_NODES // BN,),
        in_specs=[
            pl.BlockSpec((BN, D), lambda i: (i, 0)),
            pl.BlockSpec((BN, D), lambda i: (i, 0)),
            pl.BlockSpec((BN, D), lambda i: (i, 0)),
            pl.BlockSpec((BN, D), lambda i: (i, 0)),
            pl.BlockSpec((BN, D), lambda i: (i, 0)),
            full((D, HID)), full((D, HID)),
            full((1, HID)), full((HID, D)), full((1, D)),
        ],
        out_specs=pl.BlockSpec((BN, D), lambda i: (i, 0)),
        out_shape=jax.ShapeDtypeStruct((N_NODES, D), jnp.float32),
    )(V2, s0, s1, c0, c1, w1v, w1e, b1, w2, b2)


def kernel(V, E, edges, fe_W1, fe_b1, fe_W2, fe_b2, fn_W1, fn_b1, fn_W2, fn_b2):
    V2 = V[0]
    E2 = E[0]
    eidx = edges[0].astype(jnp.int32)
    sidx3 = eidx[:, 0].reshape(NW, NCH, CH)
    ridx3 = eidx[:, 1].reshape(NW, NCH, CH)

    ridx4s = eidx[:, 1].reshape(NW, NH, NCHH, CHS)
    zrow = jnp.zeros((RB + TAIL, D), jnp.float32)
    ones = jnp.ones((CHS, D), jnp.float32)

    S, R = _sc_gather(V2, sidx3, ridx3)

    emb = _tc_edge_mlp(
        S, R, E2,
        fe_W1[:D], fe_W1[D:2 * D], fe_W1[2 * D:],
        fe_b1.reshape(1, HID), fe_W2, fe_b2.reshape(1, D))

    sums_f, cnts_f = _sc_scatter(emb, ridx4s, zrow, ones)
    sums = sums_f.reshape(NC, N_NODES, D)
    cnts = cnts_f.reshape(NC, N_NODES, D)

    nodes = _tc_node_mlp(
        V2, sums[0], sums[1], cnts[0], cnts[1],
        fn_W1[:D], fn_W1[D:],
        fn_b1.reshape(1, HID), fn_W2, fn_b2.reshape(1, D))

    return (nodes[None], emb[None])


# speedup vs baseline: 1.5315x; 1.0839x over previous
import functools

import jax
import jax.numpy as jnp
from jax import lax
from jax.experimental import pallas as pl
from jax.experimental.pallas import tpu as pltpu
from jax.experimental.pallas import tpu_sc as plsc

N_NODES = 10000
N_EDGES = 320000
D = 128
HID = 256

NC = 2
NS = 16
NW = NC * NS
EPW = N_EDGES // NW
CH = 80
NCH = EPW // CH
NBUF = 4
RB = 624
TAIL = N_NODES - NS * RB

_mesh = plsc.VectorSubcoreMesh(
    core_axis_name="c", subcore_axis_name="s", num_cores=NC, num_subcores=NS)


@functools.partial(
    pl.kernel,
    out_type=(jax.ShapeDtypeStruct((N_EDGES, D), jnp.float32),
              jax.ShapeDtypeStruct((N_EDGES, D), jnp.float32)),
    mesh=_mesh,
    scratch_types=[
        pltpu.VMEM((NCH, CH), jnp.int32),
        pltpu.VMEM((NCH, CH), jnp.int32),
        pltpu.VMEM((NBUF, CH, D), jnp.float32),
        pltpu.VMEM((NBUF, CH, D), jnp.float32),
        [pltpu.SemaphoreType.DMA] * NBUF,
        [pltpu.SemaphoreType.DMA] * NBUF,
        [pltpu.SemaphoreType.DMA] * NBUF,
        [pltpu.SemaphoreType.DMA] * NBUF,
    ],
)
def _sc_gather(v_hbm, sidx_hbm, ridx_hbm,
               outs, outr,
               si_all, ri_all, bufs, bufr,
               sg, rg, sw, rw):
    cid = lax.axis_index("c")
    sid = lax.axis_index("s")
    wid = sid * NC + cid
    base = wid * EPW

    pltpu.sync_copy(sidx_hbm.at[wid], si_all)
    pltpu.sync_copy(ridx_hbm.at[wid], ri_all)

    def g_descs(c, b):
        return (pltpu.make_async_copy(v_hbm.at[si_all.at[c]], bufs.at[b], sg[b]),
                pltpu.make_async_copy(v_hbm.at[ri_all.at[c]], bufr.at[b], rg[b]))

    def w_descs(c, b):
        dst = pl.ds(base + c * CH, CH)
        return (pltpu.make_async_copy(bufs.at[b], outs.at[dst], sw[b]),
                pltpu.make_async_copy(bufr.at[b], outr.at[dst], rw[b]))

    def start_gathers(c, b):
        d1, d2 = g_descs(c, b)
        d1.start()
        d2.start()

    def visit(c, b):
        b2 = (b + 2) % NBUF

        @pl.when(c >= 2)
        def _retire():
            d1, d2 = w_descs(c - 2, b2)
            d1.wait()
            d2.wait()

        @pl.when(c + 2 < NCH)
        def _prefetch():
            start_gathers(c + 2, b2)

        d1, d2 = g_descs(c, b)
        d1.wait()
        d2.wait()
        d1, d2 = w_descs(c, b)
        d1.start()
        d2.start()

    start_gathers(0, 0)
    start_gathers(1, 1)

    def body(g, carry):
        c0 = g * NBUF
        for b in range(NBUF):
            visit(c0 + b, b)
        return carry

    lax.fori_loop(0, (NCH - 1) // NBUF, body, 0)
    visit(NCH - 1, (NCH - 1) % NBUF)

    for c in (NCH - 2, NCH - 1):
        b = c % NBUF
        d1, d2 = w_descs(c, b)
        d1.wait()
        d2.wait()


CHS = 40
NH = 2
NCHH = EPW // CHS // NH
NBS = 3


def _zero_acc(zrow_hbm, acc, sid):
    pltpu.sync_copy(zrow_hbm.at[pl.ds(0, RB)], acc.at[pl.ds(sid * RB, RB)])

    @pl.when(sid == NS - 1)
    def _zero_tail():
        pltpu.sync_copy(zrow_hbm.at[pl.ds(RB, TAIL)],
                        acc.at[pl.ds(NS * RB, TAIL)])


def _dump_acc(acc, out_hbm, cid, sid):
    obase = cid * N_NODES
    pltpu.sync_copy(acc.at[pl.ds(sid * RB, RB)],
                    out_hbm.at[pl.ds(obase + sid * RB, RB)])

    @pl.when(sid == NS - 1)
    def _dump_tail():
        pltpu.sync_copy(acc.at[pl.ds(NS * RB, TAIL)],
                        out_hbm.at[pl.ds(obase + NS * RB, TAIL)])


@functools.partial(
    pl.kernel,
    out_type=jax.ShapeDtypeStruct((NC * N_NODES, D), jnp.float32),
    mesh=_mesh,
    scratch_types=[
        pltpu.VMEM((NCHH, CHS), jnp.int32),
        pltpu.VMEM((CHS, D), jnp.float32),
        pltpu.VMEM_SHARED((N_NODES, D), jnp.float32),
        [pltpu.SemaphoreType.DMA] * 2,
    ],
)
def _sc_counts(ridx_hbm, zrow_hbm, ones_hbm, cnts_hbm,
               ri_half, ones_v, acc, ca):
    cid = lax.axis_index("c")
    sid = lax.axis_index("s")
    wid = sid * NC + cid

    pltpu.sync_copy(ones_hbm, ones_v)
    _zero_acc(zrow_hbm, acc, sid)
    plsc.subcore_barrier()

    def c_desc(c, k):
        return pltpu.make_async_copy(ones_v, acc.at[ri_half.at[c]], ca[k])

    for h in range(NH):
        pltpu.sync_copy(ridx_hbm.at[wid, h], ri_half)

        def cvisit(c, k):
            @pl.when(c >= 2)
            def _retire():
                c_desc(c - 2, k).wait()

            c_desc(c, k).start(add=True)

        def body2(g, carry, cvisit=cvisit):
            c0 = g * 2
            cvisit(c0, 0)
            cvisit(c0 + 1, 1)
            return carry

        lax.fori_loop(0, NCHH // 2, body2, 0)
        for c in range(2 * (NCHH // 2), NCHH):
            cvisit(c, c % 2)
        for c in (NCHH - 2, NCHH - 1):
            c_desc(c, c % 2).wait()

    plsc.subcore_barrier()
    _dump_acc(acc, cnts_hbm, cid, sid)


@functools.partial(
    pl.kernel,
    out_type=jax.ShapeDtypeStruct((NC * N_NODES, D), jnp.float32),
    mesh=_mesh,
    scratch_types=[
        pltpu.VMEM((NCHH, CHS), jnp.int32),
        pltpu.VMEM((NBS, CHS, D), jnp.float32),
        pltpu.VMEM_SHARED((N_NODES, D), jnp.float32),
        [pltpu.SemaphoreType.DMA] * NBS,
        [pltpu.SemaphoreType.DMA] * NBS,
    ],
)
def _sc_scatter(emb_hbm, ridx_hbm, zrow_hbm, sums_hbm,
                ri_half, bufe, acc, le, sa):
    cid = lax.axis_index("c")
    sid = lax.axis_index("s")
    wid = sid * NC + cid
    base = wid * EPW

    _zero_acc(zrow_hbm, acc, sid)
    plsc.subcore_barrier()

    def l_desc(h, c, b):
        src = pl.ds(base + (h * NCHH + c) * CHS, CHS)
        return pltpu.make_async_copy(emb_hbm.at[src], bufe.at[b], le[b])

    def a_desc(c, b):
        return pltpu.make_async_copy(bufe.at[b], acc.at[ri_half.at[c]], sa[b])

    for h in range(NH):
        pltpu.sync_copy(ridx_hbm.at[wid, h], ri_half)

        def visit(c, b, h=h):
            bp = (b + 2) % NBS

            @pl.when(c >= 1)
            def _retire():
                a_desc(c - 1, bp).wait()

            @pl.when(c + 2 < NCHH)
            def _prefetch():
                l_desc(h, c + 2, bp).start()

            l_desc(h, c, b).wait()
            a_desc(c, b).start(add=True)

        l_desc(h, 0, 0).start()
        l_desc(h, 1, 1).start()

        def body(g, carry, visit=visit):
            c0 = g * NBS
            for b in range(NBS):
                visit(c0 + b, b)
            return carry

        nfull = (NCHH - 2) // NBS
        lax.fori_loop(0, nfull, body, 0)
        for c in range(nfull * NBS, NCHH):
            visit(c, c % NBS)
        a_desc(NCHH - 1, (NCHH - 1) % NBS).wait()

    plsc.subcore_barrier()
    _dump_acc(acc, sums_hbm, cid, sid)


def _gelu(x):
    return x * 0.5 * (1.0 + lax.erf(x * 0.7071067811865476))


BE = 1280


def _edge_mlp_body(s_ref, r_ref, e_ref, w1s, w1r, w1e, b1, w2, b2, o_ref):
    x = (jnp.dot(s_ref[...], w1s[...], preferred_element_type=jnp.float32)
         + jnp.dot(r_ref[...], w1r[...], preferred_element_type=jnp.float32)
         + jnp.dot(e_ref[...], w1e[...], preferred_element_type=jnp.float32)
         + b1[...])
    h = _gelu(x)
    o_ref[...] = jnp.dot(h, w2[...], preferred_element_type=jnp.float32) + b2[...]


def _tc_edge_mlp(S, R, E2, w1s, w1r, w1e, b1, w2, b2):
    full = lambda shape: pl.BlockSpec(shape, lambda i: (0,) * len(shape))
    return pl.pallas_call(
        _edge_mlp_body,
        grid=(N_EDGES // BE,),
        in_specs=[
            pl.BlockSpec((BE, D), lambda i: (i, 0)),
            pl.BlockSpec((BE, D), lambda i: (i, 0)),
            pl.BlockSpec((BE, D), lambda i: (i, 0)),
            full((D, HID)), full((D, HID)), full((D, HID)),
            full((1, HID)), full((HID, D)), full((1, D)),
        ],
        out_specs=pl.BlockSpec((BE, D), lambda i: (i, 0)),
        out_shape=jax.ShapeDtypeStruct((N_EDGES, D), jnp.float32),
    )(S, R, E2, w1s, w1r, w1e, b1, w2, b2)


BN = 400


def _node_mlp_body(v_ref, s0, s1, c0, c1, w1v, w1e, b1, w2, b2, o_ref):
    cnt = c0[...][:, :1] + c1[...][:, :1]
    es = (s0[...] + s1[...]) / jnp.maximum(cnt, 1.0)
    x = (jnp.dot(v_ref[...], w1v[...], preferred_element_type=jnp.float32)
         + jnp.dot(es, w1e[...], preferred_element_type=jnp.float32)
         + b1[...])
    h = _gelu(x)
    o_ref[...] = jnp.dot(h, w2[...], preferred_element_type=jnp.float32) + b2[...]


def _tc_node_mlp(V2, s0, s1, c0, c1, w1v, w1e, b1, w2, b2):
    full = lambda shape: pl.BlockSpec(shape, lambda i: (0,) * len(shape))
    return pl.pallas_call(
        _node_mlp_body,
        grid=(N_NODES // BN,),
        in_specs=[
            pl.BlockSpec((BN, D), lambda i: (i, 0)),
            pl.BlockSpec((BN, D), lambda i: (i, 0)),
            pl.BlockSpec((BN, D), lambda i: (i, 0)),
            pl.BlockSpec((BN, D), lambda i: (i, 0)),
            pl.BlockSpec((BN, D), lambda i: (i, 0)),
            full((D, HID)), full((D, HID)),
            full((1, HID)), full((HID, D)), full((1, D)),
        ],
        out_specs=pl.BlockSpec((BN, D), lambda i: (i, 0)),
        out_shape=jax.ShapeDtypeStruct((N_NODES, D), jnp.float32),
    )(V2, s0, s1, c0, c1, w1v, w1e, b1, w2, b2)


def kernel(V, E, edges, fe_W1, fe_b1, fe_W2, fe_b2, fn_W1, fn_b1, fn_W2, fn_b2):
    V2 = V[0]
    E2 = E[0]
    eidx = edges[0].astype(jnp.int32)
    sidx3 = eidx[:, 0].reshape(NW, NCH, CH)
    ridx3 = eidx[:, 1].reshape(NW, NCH, CH)

    ridx4s = eidx[:, 1].reshape(NW, NH, NCHH, CHS)
    zrow = jnp.zeros((RB + TAIL, D), jnp.float32)
    ones = jnp.ones((CHS, D), jnp.float32)

    S, R = _sc_gather(V2, sidx3, ridx3)

    emb = _tc_edge_mlp(
        S, R, E2,
        fe_W1[:D], fe_W1[D:2 * D], fe_W1[2 * D:],
        fe_b1.reshape(1, HID), fe_W2, fe_b2.reshape(1, D))

    cnts_f = _sc_counts(ridx4s, zrow, ones)
    sums_f = _sc_scatter(emb, ridx4s, zrow)
    sums = sums_f.reshape(NC, N_NODES, D)
    cnts = cnts_f.reshape(NC, N_NODES, D)

    nodes = _tc_node_mlp(
        V2, sums[0], sums[1], cnts[0], cnts[1],
        fn_W1[:D], fn_W1[D:],
        fn_b1.reshape(1, HID), fn_W2, fn_b2.reshape(1, D))

    return (nodes[None], emb[None])
